# baseline (speedup 1.0000x reference)
import jax
import jax.numpy as jnp
from jax.experimental import pallas as pl
from jax.experimental.pallas import tpu as pltpu

B, S, D = 8, 197, 768
H = 12
P = 196
KW = 48
INV_TEMP = 10.0
MLP_DIM = 3072
HEAD_DIM = D // H
SCALE = HEAD_DIM ** -0.5
NEG = -1e9


BPG = 4


def _block_body(x_ref, wq_ref, wk_ref, bqk_ref, pos_ref, wqkv_ref, bqkv_ref,
                wproj_ref, bproj_ref, ln_ref, wm1_ref, bm1_ref, wm2_ref,
                bm2_ref, out_ref, attn_ref):
    for bi in range(BPG):
        _one_batch(bi, x_ref, wq_ref, wk_ref, bqk_ref, pos_ref, wqkv_ref,
                   bqkv_ref, wproj_ref, bproj_ref, ln_ref, wm1_ref, bm1_ref,
                   wm2_ref, bm2_ref, out_ref, attn_ref)


def _one_batch(bi, x_ref, wq_ref, wk_ref, bqk_ref, pos_ref, wqkv_ref, bqkv_ref,
               wproj_ref, bproj_ref, ln_ref, wm1_ref, bm1_ref, wm2_ref,
               bm2_ref, out_ref, attn_ref):
    f32 = jnp.float32
    bf16 = jnp.bfloat16
    xb = x_ref[bi]
    ln = ln_ref[...]

    m = jnp.mean(xb, axis=-1, keepdims=True)
    xc = xb - m
    v = jnp.mean(xc * xc, axis=-1, keepdims=True)
    xn = xc * jax.lax.rsqrt(v + 1e-5) * ln[0:1] + ln[1:2]
    xn16 = xn
    xp16 = xn16[1:]

    bqk = bqk_ref[...]
    q = jnp.dot(xp16, wq_ref[...], preferred_element_type=f32) + bqk[0:1]
    k = jnp.dot(xp16, wk_ref[...], preferred_element_type=f32) + bqk[1:2]
    q = q * jax.lax.rsqrt(jnp.maximum(jnp.sum(q * q, -1, keepdims=True), 1e-24))
    k = k * jax.lax.rsqrt(jnp.maximum(jnp.sum(k * k, -1, keepdims=True), 1e-24))
    s_t = jax.lax.dot_general(k, q,
                              (((1,), (1,)), ((), ())),
                              preferred_element_type=f32)
    ri = jax.lax.broadcasted_iota(jnp.int32, (P, P), 0)
    ci = jax.lax.broadcasted_iota(jnp.int32, (P, P), 1)
    s_t = jnp.where(ri == ci, NEG, s_t + pos_ref[...])
    s_t = s_t * INV_TEMP

    m0 = jnp.max(s_t, axis=0, keepdims=True)
    cur = jnp.where(s_t >= m0, -1e30, s_t)
    acc = jnp.ones((1, P), f32)
    t = m0
    for _ in range(KW - 1):
        t = jnp.max(cur, axis=0, keepdims=True)
        acc = acc + jnp.exp(t - m0)
        cur = jnp.where(cur >= t, -1e30, cur)
    lse = m0 + jnp.log(acc)
    bias_t = jnp.where(s_t >= t, jnp.maximum(s_t - lse, -10.0), NEG)

    qkv = jnp.dot(xn16, wqkv_ref[...], preferred_element_type=f32) + bqkv_ref[...]

    hh = jax.lax.broadcasted_iota(jnp.int32, (H, D), 0)
    cc = jax.lax.broadcasted_iota(jnp.int32, (H, D), 1)
    blk = jnp.where(cc // HEAD_DIM == hh, 1.0, 0.0).astype(f32)
    qcb = (qkv[0:1, 0:D] * SCALE) * blk
    kfull = qkv[:, D:2 * D]
    vfull = qkv[:, 2 * D:3 * D]
    lcs = jax.lax.dot_general(qcb, kfull, (((1,), (1,)), ((), ())),
                              preferred_element_type=f32)
    lcs = lcs - jnp.max(lcs, axis=1, keepdims=True)
    ecs = jnp.exp(lcs)
    acs = ecs / jnp.sum(ecs, axis=1, keepdims=True)
    ocf = jnp.dot(acs, vfull, preferred_element_type=f32)
    attn_ref[0:1, :] = jnp.sum(ocf * blk, axis=0, keepdims=True)

    for h in range(H):
        qh = qkv[1:, h * HEAD_DIM:(h + 1) * HEAD_DIM] * SCALE
        kh = qkv[1:, D + h * HEAD_DIM:D + (h + 1) * HEAD_DIM]
        vh = qkv[1:, 2 * D + h * HEAD_DIM:2 * D + (h + 1) * HEAD_DIM]
        logits = jax.lax.dot_general(kh, qh, (((1,), (1,)), ((), ())),
                                     preferred_element_type=f32) + bias_t
        e = jnp.exp(logits)
        a = e * (1.0 / jnp.sum(e, axis=0, keepdims=True))
        op = jax.lax.dot_general(a, vh, (((0,), (0,)), ((), ())),
                                 preferred_element_type=f32)
        attn_ref[1:, h * HEAD_DIM:(h + 1) * HEAD_DIM] = op
    attn = attn_ref[...]

    y = xb + jnp.dot(attn, wproj_ref[...], preferred_element_type=f32) + bproj_ref[...]

    m2 = jnp.mean(y, axis=-1, keepdims=True)
    yc = y - m2
    v2 = jnp.mean(yc * yc, axis=-1, keepdims=True)
    yn16 = yc * jax.lax.rsqrt(v2 + 1e-5) * ln[2:3] + ln[3:4]
    hmid = jnp.dot(yn16, wm1_ref[...], preferred_element_type=f32) + bm1_ref[...]
    g = 0.5 * hmid * (1.0 + jax.lax.erf(hmid * 0.7071067811865476))
    out2 = jnp.dot(g, wm2_ref[...], preferred_element_type=f32) + bm2_ref[...]
    out_ref[bi] = y + out2


def kernel(x, Wq, bq, Wk, bk, pos_bias, Wqkv, bqkv, Wproj, bproj,
           ln1_g, ln1_b, ln2_g, ln2_b, Wm1, bm1, Wm2, bm2):
    bf16 = jnp.bfloat16
    bqk = jnp.stack([bq, bk])
    ln = jnp.stack([ln1_g, ln1_b, ln2_g, ln2_b])
    full = lambda shp: pl.BlockSpec(shp, lambda b: (0,) * len(shp))
    grid_spec = pl.GridSpec(
        grid=(B // BPG,),
        in_specs=[
            pl.BlockSpec((BPG, S, D), lambda b: (b, 0, 0)),
            full((D, D)),
            full((D, D)),
            full((2, D)),
            full((P, P)),
            full((D, 3 * D)),
            full((1, 3 * D)),
            full((D, D)),
            full((1, D)),
            full((4, D)),
            full((D, MLP_DIM)),
            full((1, MLP_DIM)),
            full((MLP_DIM, D)),
            full((1, D)),
        ],
        out_specs=pl.BlockSpec((BPG, S, D), lambda b: (b, 0, 0)),
        scratch_shapes=[pltpu.VMEM((S, D), jnp.float32)],
    )
    return pl.pallas_call(
        _block_body,
        grid_spec=grid_spec,
        out_shape=jax.ShapeDtypeStruct((B, S, D), jnp.float32),
    )(x, Wq, Wk, bqk, pos_bias,
      Wqkv, bqkv.reshape(1, -1), Wproj,
      bproj.reshape(1, -1), ln, Wm1, bm1.reshape(1, -1),
      Wm2, bm2.reshape(1, -1))


# speedup vs baseline: 1.0196x; 1.0196x over previous
import jax
import jax.numpy as jnp
from jax.experimental import pallas as pl
from jax.experimental.pallas import tpu as pltpu

B, S, D = 8, 197, 768
H = 12
P = 196
KW = 48
INV_TEMP = 10.0
MLP_DIM = 3072
HEAD_DIM = D // H
SCALE = HEAD_DIM ** -0.5
NEG = -1e9


BPG = 4


def _block_body(x_ref, wq_ref, wk_ref, bqk_ref, pos_ref, wqkv_ref, bqkv_ref,
                wproj_ref, bproj_ref, ln_ref, wm1_ref, bm1_ref, wm2_ref,
                bm2_ref, out_ref):
    for bi in range(BPG):
        _one_batch(bi, x_ref, wq_ref, wk_ref, bqk_ref, pos_ref, wqkv_ref,
                   bqkv_ref, wproj_ref, bproj_ref, ln_ref, wm1_ref, bm1_ref,
                   wm2_ref, bm2_ref, out_ref)


def _one_batch(bi, x_ref, wq_ref, wk_ref, bqk_ref, pos_ref, wqkv_ref, bqkv_ref,
               wproj_ref, bproj_ref, ln_ref, wm1_ref, bm1_ref, wm2_ref,
               bm2_ref, out_ref):
    f32 = jnp.float32
    bf16 = jnp.bfloat16
    xb = x_ref[bi]
    ln = ln_ref[...]

    m = jnp.mean(xb, axis=-1, keepdims=True)
    xc = xb - m
    v = jnp.mean(xc * xc, axis=-1, keepdims=True)
    xn = xc * jax.lax.rsqrt(v + 1e-5) * ln[0:1] + ln[1:2]
    xn16 = xn
    xp16 = xn16[1:]

    bqk = bqk_ref[...]
    q = jnp.dot(xp16, wq_ref[...], preferred_element_type=f32) + bqk[0:1]
    k = jnp.dot(xp16, wk_ref[...], preferred_element_type=f32) + bqk[1:2]
    q = q * jax.lax.rsqrt(jnp.maximum(jnp.sum(q * q, -1, keepdims=True), 1e-24))
    k = k * jax.lax.rsqrt(jnp.maximum(jnp.sum(k * k, -1, keepdims=True), 1e-24))
    s_t = jax.lax.dot_general(k, q,
                              (((1,), (1,)), ((), ())),
                              preferred_element_type=f32)
    ri = jax.lax.broadcasted_iota(jnp.int32, (P, P), 0)
    ci = jax.lax.broadcasted_iota(jnp.int32, (P, P), 1)
    s_t = jnp.where(ri == ci, NEG, s_t + pos_ref[...])
    s_t = s_t * INV_TEMP

    m0 = jnp.max(s_t, axis=0, keepdims=True)
    cur = jnp.where(s_t >= m0, -1e30, s_t)
    acc = jnp.ones((1, P), f32)
    t = m0
    for _ in range(KW - 1):
        t = jnp.max(cur, axis=0, keepdims=True)
        acc = acc + jnp.exp(t - m0)
        cur = jnp.where(cur >= t, -1e30, cur)
    lse = m0 + jnp.log(acc)
    bias_t = jnp.where(s_t >= t, jnp.maximum(s_t - lse, -10.0), NEG)

    qkv = jnp.dot(xn16, wqkv_ref[...], preferred_element_type=f32) + bqkv_ref[...]

    hh = jax.lax.broadcasted_iota(jnp.int32, (H, D), 0)
    cc = jax.lax.broadcasted_iota(jnp.int32, (H, D), 1)
    blk = jnp.where(cc // HEAD_DIM == hh, 1.0, 0.0).astype(f32)
    qcb = (qkv[0:1, 0:D] * SCALE) * blk
    kfull = qkv[:, D:2 * D]
    vfull = qkv[:, 2 * D:3 * D]
    lcs = jax.lax.dot_general(qcb, kfull, (((1,), (1,)), ((), ())),
                              preferred_element_type=f32)
    lcs = lcs - jnp.max(lcs, axis=1, keepdims=True)
    ecs = jnp.exp(lcs)
    acs = ecs / jnp.sum(ecs, axis=1, keepdims=True)
    ocf = jnp.dot(acs, vfull, preferred_element_type=f32)
    oc_row = jnp.sum(ocf * blk, axis=0, keepdims=True)

    outs = []
    for h in range(H):
        qh = qkv[1:, h * HEAD_DIM:(h + 1) * HEAD_DIM] * SCALE
        kh = qkv[1:, D + h * HEAD_DIM:D + (h + 1) * HEAD_DIM]
        vh = qkv[1:, 2 * D + h * HEAD_DIM:2 * D + (h + 1) * HEAD_DIM]
        logits = jax.lax.dot_general(kh, qh, (((1,), (1,)), ((), ())),
                                     preferred_element_type=f32) + bias_t
        e = jnp.exp(logits)
        a = e * (1.0 / jnp.sum(e, axis=0, keepdims=True))
        op = jax.lax.dot_general(a, vh, (((0,), (0,)), ((), ())),
                                 preferred_element_type=f32)
        outs.append(op)
    attn = jnp.concatenate(
        [oc_row, jnp.concatenate(outs, axis=1)], axis=0)

    y = xb + jnp.dot(attn, wproj_ref[...], preferred_element_type=f32) + bproj_ref[...]

    m2 = jnp.mean(y, axis=-1, keepdims=True)
    yc = y - m2
    v2 = jnp.mean(yc * yc, axis=-1, keepdims=True)
    yn16 = yc * jax.lax.rsqrt(v2 + 1e-5) * ln[2:3] + ln[3:4]
    hmid = jnp.dot(yn16, wm1_ref[...], preferred_element_type=f32) + bm1_ref[...]
    g = 0.5 * hmid * (1.0 + jax.lax.erf(hmid * 0.7071067811865476))
    out2 = jnp.dot(g, wm2_ref[...], preferred_element_type=f32) + bm2_ref[...]
    out_ref[bi] = y + out2


def kernel(x, Wq, bq, Wk, bk, pos_bias, Wqkv, bqkv, Wproj, bproj,
           ln1_g, ln1_b, ln2_g, ln2_b, Wm1, bm1, Wm2, bm2):
    bf16 = jnp.bfloat16
    bqk = jnp.stack([bq, bk])
    ln = jnp.stack([ln1_g, ln1_b, ln2_g, ln2_b])
    full = lambda shp: pl.BlockSpec(shp, lambda b: (0,) * len(shp))
    grid_spec = pl.GridSpec(
        grid=(B // BPG,),
        in_specs=[
            pl.BlockSpec((BPG, S, D), lambda b: (b, 0, 0)),
            full((D, D)),
            full((D, D)),
            full((2, D)),
            full((P, P)),
            full((D, 3 * D)),
            full((1, 3 * D)),
            full((D, D)),
            full((1, D)),
            full((4, D)),
            full((D, MLP_DIM)),
            full((1, MLP_DIM)),
            full((MLP_DIM, D)),
            full((1, D)),
        ],
        out_specs=pl.BlockSpec((BPG, S, D), lambda b: (b, 0, 0)),
    )
    return pl.pallas_call(
        _block_body,
        grid_spec=grid_spec,
        out_shape=jax.ShapeDtypeStruct((B, S, D), jnp.float32),
    )(x, Wq, Wk, bqk, pos_bias,
      Wqkv, bqkv.reshape(1, -1), Wproj,
      bproj.reshape(1, -1), ln, Wm1, bm1.reshape(1, -1),
      Wm2, bm2.reshape(1, -1))
